# baseline (speedup 1.0000x reference)
import functools

import jax
import jax.numpy as jnp
from jax import lax
from jax.experimental import pallas as pl
from jax.experimental.pallas import tpu as pltpu
from jax.experimental.pallas import tpu_sc as plsc

N_NODES = 10000
D = 128
NC = 2
NS = 16
NW = NC * NS
CK = 128
SB = 6
NSB = 13
NCH = NSB * SB
ESB = SB * CK
EPW = NCH * CK
NREM = 4
ACC_ROWS = 10240
ROWS_PER_TILE = ACC_ROWS // NS


def _sc_message_passing(feat_hbm, edge_hbm, part_hbm,
                        eidx_sb, rows0, rows1, acc, g0, g1, i_sem):
    c = lax.axis_index("c")
    s = lax.axis_index("s")
    w = s * NC + c
    ebase = w * EPW

    pltpu.async_copy(edge_hbm.at[:, pl.ds(ebase, ESB)], eidx_sb.at[0], i_sem)

    def _zero_row(i, carry):
        rows0[i >> 3, pl.ds((i & 7) * 16, 16)] = jnp.zeros((16,), jnp.float32)
        return carry
    lax.fori_loop(0, 128 * 8, _zero_row, 0)
    zbase = s * ROWS_PER_TILE
    for k in range(ROWS_PER_TILE // 128):
        pltpu.sync_copy(rows0, acc.at[pl.ds(zbase + k * 128, 128)])
    plsc.subcore_barrier()

    pltpu.make_async_copy(edge_hbm.at[:, pl.ds(ebase, ESB)],
                          eidx_sb.at[0], i_sem).wait()
    pltpu.async_copy(feat_hbm.at[eidx_sb.at[0, 0, pl.ds(0, CK)]], rows0, g0)
    pltpu.async_copy(feat_hbm.at[eidx_sb.at[0, 0, pl.ds(CK, CK)]], rows1, g1)

    rows = (rows0, rows1)
    gsem = (g0, g1)

    def _super_block(sbi, carry):
        slot = sbi & 1
        nslot = 1 - slot
        last = sbi == NSB - 1
        nb = ebase + (sbi + 1) * ESB
        @pl.when(jnp.logical_not(last))
        def _():
            pltpu.async_copy(edge_hbm.at[:, pl.ds(nb, ESB)],
                             eidx_sb.at[nslot], i_sem)
        for k in range(SB):
            if k == SB - 2:
                @pl.when(jnp.logical_not(last))
                def _():
                    pltpu.make_async_copy(edge_hbm.at[:, pl.ds(nb, ESB)],
                                          eidx_sb.at[nslot], i_sem).wait()
            p = k & 1
            pltpu.make_async_copy(
                feat_hbm.at[eidx_sb.at[slot, 0, pl.ds(k * CK, CK)]],
                rows[p], gsem[p]).wait()
            pltpu.sync_copy(
                rows[p], acc.at[eidx_sb.at[slot, 1, pl.ds(k * CK, CK)]],
                add=True)
            if k < SB - 2:
                pltpu.async_copy(
                    feat_hbm.at[eidx_sb.at[slot, 0, pl.ds((k + 2) * CK, CK)]],
                    rows[p], gsem[p])
            else:
                @pl.when(jnp.logical_not(last))
                def _():
                    pltpu.async_copy(
                        feat_hbm.at[eidx_sb.at[nslot, 0,
                                               pl.ds((k + 2 - SB) * CK, CK)]],
                        rows[p], gsem[p])
        return carry
    lax.fori_loop(0, NSB, _super_block, 0)

    @pl.when(w < NREM)
    def _():
        rem = (NW * NCH + w) * CK
        pltpu.sync_copy(edge_hbm.at[:, pl.ds(rem, CK)],
                        eidx_sb.at[0, :, pl.ds(0, CK)])
        pltpu.async_copy(feat_hbm.at[eidx_sb.at[0, 0, pl.ds(0, CK)]],
                         rows0, g0)
        pltpu.make_async_copy(feat_hbm.at[eidx_sb.at[0, 0, pl.ds(0, CK)]],
                              rows0, g0).wait()
        pltpu.sync_copy(rows0, acc.at[eidx_sb.at[0, 1, pl.ds(0, CK)]],
                        add=True)

    plsc.subcore_barrier()
    pltpu.sync_copy(acc.at[pl.ds(zbase, ROWS_PER_TILE)],
                    part_hbm.at[c, pl.ds(zbase, ROWS_PER_TILE)])


@functools.partial(
    pl.kernel,
    out_type=jax.ShapeDtypeStruct((NC, ACC_ROWS, D), jnp.float32),
    mesh=plsc.VectorSubcoreMesh(core_axis_name="c", subcore_axis_name="s",
                                num_cores=NC, num_subcores=NS),
    scratch_types=[
        pltpu.VMEM((2, 2, ESB), jnp.int32),
        pltpu.VMEM((CK, D), jnp.float32),
        pltpu.VMEM((CK, D), jnp.float32),
        pltpu.VMEM_SHARED((ACC_ROWS, D), jnp.float32),
        pltpu.SemaphoreType.DMA,
        pltpu.SemaphoreType.DMA,
        pltpu.SemaphoreType.DMA,
    ],
)
def _sc_kernel(feat_hbm, edge_hbm, part_hbm,
               eidx_sb, rows0, rows1, acc, g0, g1, i_sem):
    _sc_message_passing(feat_hbm, edge_hbm, part_hbm,
                        eidx_sb, rows0, rows1, acc, g0, g1, i_sem)


def _tc_linear_body(p_ref, w_ref, b_ref, o_ref):
    h = p_ref[0] + p_ref[1]
    o_ref[...] = lax.dot_general(
        h, w_ref[...], (((1,), (1,)), ((), ())),
        preferred_element_type=jnp.float32) + b_ref[...]


def _tc_linear(partials, W, b2):
    blk = 10000
    return pl.pallas_call(
        _tc_linear_body,
        grid=(N_NODES // blk,),
        in_specs=[
            pl.BlockSpec((NC, blk, D), lambda i: (0, i, 0)),
            pl.BlockSpec((D, D), lambda i: (0, 0)),
            pl.BlockSpec((1, D), lambda i: (0, 0)),
        ],
        out_specs=pl.BlockSpec((blk, D), lambda i: (i, 0)),
        out_shape=jax.ShapeDtypeStruct((N_NODES, D), jnp.float32),
    )(partials, W, b2)


def kernel(features, edge_index, W, b):
    partials = _sc_kernel(features, edge_index.astype(jnp.int32))
    return _tc_linear(partials, W, b.reshape(1, D))


# speedup vs baseline: 1.0097x; 1.0097x over previous
import functools

import jax
import jax.numpy as jnp
from jax import lax
from jax.experimental import pallas as pl
from jax.experimental.pallas import tpu as pltpu
from jax.experimental.pallas import tpu_sc as plsc

N_NODES = 10000
D = 128
NC = 2
NS = 16
NW = NC * NS
CK = 128
SB = 6
NSB = 13
NCH = NSB * SB
ESB = SB * CK
EPW = NCH * CK
NREM = 4
ACC_ROWS = 10240
ROWS_PER_TILE = ACC_ROWS // NS


def _sc_message_passing(feat_hbm, edge_hbm, part_hbm,
                        eidx_sb, rows0, rows1, acc, g0, g1, i_sem):
    c = lax.axis_index("c")
    s = lax.axis_index("s")
    w = s * NC + c
    ebase = w * EPW

    pltpu.async_copy(edge_hbm.at[:, pl.ds(ebase, ESB)], eidx_sb.at[0], i_sem)

    def _zero_row(i, carry):
        rows0[i >> 3, pl.ds((i & 7) * 16, 16)] = jnp.zeros((16,), jnp.float32)
        return carry
    lax.fori_loop(0, 128 * 8, _zero_row, 0)
    zbase = s * ROWS_PER_TILE
    for k in range(ROWS_PER_TILE // 128):
        pltpu.sync_copy(rows0, acc.at[pl.ds(zbase + k * 128, 128)])
    plsc.subcore_barrier()

    pltpu.make_async_copy(edge_hbm.at[:, pl.ds(ebase, ESB)],
                          eidx_sb.at[0], i_sem).wait()
    pltpu.async_copy(feat_hbm.at[eidx_sb.at[0, 0, pl.ds(0, CK)]], rows0, g0)
    pltpu.async_copy(feat_hbm.at[eidx_sb.at[0, 0, pl.ds(CK, CK)]], rows1, g1)

    rows = (rows0, rows1)
    gsem = (g0, g1)

    def _super_block(sbi, carry):
        slot = sbi & 1
        nslot = 1 - slot
        last = sbi == NSB - 1
        nb = ebase + (sbi + 1) * ESB
        @pl.when(jnp.logical_not(last))
        def _():
            pltpu.async_copy(edge_hbm.at[:, pl.ds(nb, ESB)],
                             eidx_sb.at[nslot], i_sem)
        for k in range(SB):
            if k == SB - 2:
                @pl.when(jnp.logical_not(last))
                def _():
                    pltpu.make_async_copy(edge_hbm.at[:, pl.ds(nb, ESB)],
                                          eidx_sb.at[nslot], i_sem).wait()
            p = k & 1
            pltpu.make_async_copy(
                feat_hbm.at[eidx_sb.at[slot, 0, pl.ds(k * CK, CK)]],
                rows[p], gsem[p]).wait()
            pltpu.sync_copy(
                rows[p], acc.at[eidx_sb.at[slot, 1, pl.ds(k * CK, CK)]],
                add=True)
            if k < SB - 2:
                pltpu.async_copy(
                    feat_hbm.at[eidx_sb.at[slot, 0, pl.ds((k + 2) * CK, CK)]],
                    rows[p], gsem[p])
            else:
                @pl.when(jnp.logical_not(last))
                def _():
                    pltpu.async_copy(
                        feat_hbm.at[eidx_sb.at[nslot, 0,
                                               pl.ds((k + 2 - SB) * CK, CK)]],
                        rows[p], gsem[p])
        return carry
    lax.fori_loop(0, NSB, _super_block, 0)

    @pl.when(w < NREM)
    def _():
        rem = (NW * NCH + w) * CK
        pltpu.sync_copy(edge_hbm.at[:, pl.ds(rem, CK)],
                        eidx_sb.at[0, :, pl.ds(0, CK)])
        pltpu.async_copy(feat_hbm.at[eidx_sb.at[0, 0, pl.ds(0, CK)]],
                         rows0, g0)
        pltpu.make_async_copy(feat_hbm.at[eidx_sb.at[0, 0, pl.ds(0, CK)]],
                              rows0, g0).wait()
        pltpu.sync_copy(rows0, acc.at[eidx_sb.at[0, 1, pl.ds(0, CK)]],
                        add=True)

    plsc.subcore_barrier()
    pltpu.sync_copy(acc.at[pl.ds(zbase, ROWS_PER_TILE)],
                    part_hbm.at[c, pl.ds(zbase, ROWS_PER_TILE)])


@functools.partial(
    pl.kernel,
    out_type=jax.ShapeDtypeStruct((NC, ACC_ROWS, D), jnp.float32),
    mesh=plsc.VectorSubcoreMesh(core_axis_name="c", subcore_axis_name="s",
                                num_cores=NC, num_subcores=NS),
    scratch_types=[
        pltpu.VMEM((2, 2, ESB), jnp.int32),
        pltpu.VMEM((CK, D), jnp.float32),
        pltpu.VMEM((CK, D), jnp.float32),
        pltpu.VMEM_SHARED((ACC_ROWS, D), jnp.float32),
        pltpu.SemaphoreType.DMA,
        pltpu.SemaphoreType.DMA,
        pltpu.SemaphoreType.DMA,
    ],
)
def _sc_kernel(feat_hbm, edge_hbm, part_hbm,
               eidx_sb, rows0, rows1, acc, g0, g1, i_sem):
    _sc_message_passing(feat_hbm, edge_hbm, part_hbm,
                        eidx_sb, rows0, rows1, acc, g0, g1, i_sem)


def _tc_linear_body(p_ref, w_ref, b_ref, o_ref):
    h = p_ref[0] + p_ref[1]
    o_ref[...] = lax.dot_general(
        h, w_ref[...], (((1,), (1,)), ((), ())),
        preferred_element_type=jnp.float32) + b_ref[...]


def _tc_linear(partials, W, b2):
    blk = 5000
    return pl.pallas_call(
        _tc_linear_body,
        grid=(N_NODES // blk,),
        in_specs=[
            pl.BlockSpec((NC, blk, D), lambda i: (0, i, 0)),
            pl.BlockSpec((D, D), lambda i: (0, 0)),
            pl.BlockSpec((1, D), lambda i: (0, 0)),
        ],
        out_specs=pl.BlockSpec((blk, D), lambda i: (i, 0)),
        out_shape=jax.ShapeDtypeStruct((N_NODES, D), jnp.float32),
    )(partials, W, b2)


def kernel(features, edge_index, W, b):
    partials = _sc_kernel(features, edge_index.astype(jnp.int32))
    return _tc_linear(partials, W, b.reshape(1, D))
